# baseline (speedup 1.0000x reference)
import jax
import jax.numpy as jnp
from jax import lax
from jax.experimental import pallas as pl
from jax.experimental.pallas import tpu as pltpu
import jax.experimental.pallas.tpu_sc as plsc

N = 10000
E = 320000
D = 128
NC, NS = 2, 16
NW = NC * NS
RL = 64
SL = 128
GROWS = 5120
SROWS = 2560
GCH = 16
SCH = 8
G0 = 96
G1 = (GROWS // NS) - G0
NPAD = 10240
STRIPE = NPAD // NS
NREL = 5
SRC_OF = (0, 0, 1, 1, 2)
DST_OF = (1, 2, 0, 2, 1)


def _sc_body(xa, xe, xf,
             si0, si1, si2, si3, si4,
             di0, di1, di2, di3, di4,
             zeros_h, ones_h,
             s_out, c_out,
             s_sh, siv, div, bufa, bufb, sema0, sema1, semb0, semb1):
    cid = lax.axis_index("c")
    sid = lax.axis_index("s")
    wid = cid * NS + sid
    xs = (xa, xe, xf)
    sis = (si0, si1, si2, si3, si4)
    dis = (di0, di1, di2, di3, di4)

    nfull = STRIPE // SL
    off0 = sid * STRIPE
    is0 = cid == 0
    gbase = jnp.where(is0, sid * G0, NS * G0 + sid * G1)
    sbase = jnp.where(is0, sid * (G0 // 2),
                      NS * (G0 // 2) + sid * (G1 // 2))
    nchunkw = jnp.where(is0, G0 // GCH, G1 // GCH)

    def zero_acc():
        pltpu.sync_copy(zeros_h, bufa)

        def zstep(j, carry):
            pltpu.sync_copy(bufa, s_sh.at[pl.ds(off0 + j * SL, SL)])
            return carry

        lax.fori_loop(0, nfull, zstep, 0)
        plsc.subcore_barrier()

    def flush_acc(dst):
        def fstep(j, carry):
            off = off0 + j * SL
            pltpu.sync_copy(s_sh.at[pl.ds(off, SL)], bufa)
            pltpu.sync_copy(bufa, dst.at[pl.ds(off, SL)])
            return carry

        lax.fori_loop(0, nfull, fstep, 0)
        plsc.subcore_barrier()

    for r in range(NREL):
        src = xs[SRC_OF[r]]
        si = sis[r]
        di = dis[r]

        zero_acc()

        def gather_pair(g0, buf, s0, s1):
            da = pltpu.async_copy(src.at[siv.at[g0]], buf.at[pl.ds(0, RL)],
                                  s0)
            db = pltpu.async_copy(src.at[siv.at[g0 + 1]],
                                  buf.at[pl.ds(RL, RL)], s1)
            return da, db

        def wait_pair(buf, s0, s1):
            pltpu.make_async_copy(src.at[siv.at[0]], buf.at[pl.ds(0, RL)],
                                  s0).wait()
            pltpu.make_async_copy(src.at[siv.at[0]], buf.at[pl.ds(RL, RL)],
                                  s1).wait()

        def chunk(ch, carry):
            pltpu.sync_copy(si.at[pl.ds(gbase + ch * GCH, GCH)], siv)
            pltpu.sync_copy(di.at[pl.ds(sbase + ch * SCH, SCH)], div)
            gather_pair(0, bufa, sema0, sema1)

            def step(jj, c2):
                gather_pair(4 * jj + 2, bufb, semb0, semb1)
                wait_pair(bufa, sema0, sema1)
                pltpu.sync_copy(bufa, s_sh.at[div.at[2 * jj]], add=True)

                @pl.when(jj < SCH // 2 - 1)
                def _():
                    gather_pair(4 * jj + 4, bufa, sema0, sema1)

                wait_pair(bufb, semb0, semb1)
                pltpu.sync_copy(bufb, s_sh.at[div.at[2 * jj + 1]], add=True)
                return c2

            lax.fori_loop(0, SCH // 2, step, 0)
            return carry

        lax.fori_loop(0, nchunkw, chunk, 0)
        plsc.subcore_barrier()
        flush_acc(s_out.at[cid, r])

        zero_acc()
        pltpu.sync_copy(ones_h, bufa)

        def cchunk(ch, carry):
            pltpu.sync_copy(di.at[pl.ds(sbase + ch * SCH, SCH)], div)

            def cstep(k, c2):
                pltpu.sync_copy(bufa, s_sh.at[div.at[k]], add=True)
                return c2

            lax.fori_loop(0, SCH, cstep, 0)
            return carry

        lax.fori_loop(0, nchunkw, cchunk, 0)
        plsc.subcore_barrier()
        flush_acc(c_out.at[cid, r])


def _make_sc_kernel():
    mesh = plsc.VectorSubcoreMesh(core_axis_name="c", subcore_axis_name="s",
                                  num_cores=NC, num_subcores=NS)
    return pl.kernel(
        _sc_body,
        out_type=(
            jax.ShapeDtypeStruct((NC, NREL, NPAD, D), jnp.float32),
            jax.ShapeDtypeStruct((NC, NREL, NPAD, D), jnp.float32),
        ),
        mesh=mesh,
        scratch_types=(
            pltpu.VMEM_SHARED((NPAD, D), jnp.float32),
            pltpu.VMEM((GCH, RL), jnp.int32),
            pltpu.VMEM((SCH, SL), jnp.int32),
            pltpu.VMEM((SL, D), jnp.float32),
            pltpu.VMEM((SL, D), jnp.float32),
            pltpu.SemaphoreType.DMA,
            pltpu.SemaphoreType.DMA,
            pltpu.SemaphoreType.DMA,
            pltpu.SemaphoreType.DMA,
        ),
    )


def _tc_body(xa, xe, xf, sp, cp, llw, llb, lrw, oa, oe, of):
    x = (xa, xe, xf)
    h = []
    for r in range(NREL):
        xd = x[DST_OF[r]][...]
        s = sp[0, r] + sp[1, r]
        c = cp[0, r, :, 0] + cp[1, r, :, 0]
        c = jnp.maximum(c, 1.0)
        out = lax.dot_general(xd, llw[r], (((1,), (1,)), ((), ())),
                              precision=lax.Precision.HIGHEST,
                              preferred_element_type=jnp.float32)
        out = out + llb[r][None, :]
        agg = lax.dot_general(s, lrw[r], (((1,), (1,)), ((), ())),
                              precision=lax.Precision.HIGHEST,
                              preferred_element_type=jnp.float32)
        out = out + agg / c[:, None]
        nrm = jnp.maximum(jnp.sqrt(jnp.sum(out * out, axis=-1, keepdims=True)),
                          1e-12)
        h.append(out / nrm)
    oa[...] = h[2]
    oe[...] = h[0] + h[4]
    of[...] = h[1] + h[3]


def _tc_call(xa, xe, xf, s_part, c_part, ll_w, ll_b, lr_w):
    blk = 1280
    grid = (NPAD // blk,)
    full3 = lambda i: (0, 0, 0)
    return pl.pallas_call(
        _tc_body,
        grid=grid,
        in_specs=[
            pl.BlockSpec((blk, D), lambda i: (i, 0)),
            pl.BlockSpec((blk, D), lambda i: (i, 0)),
            pl.BlockSpec((blk, D), lambda i: (i, 0)),
            pl.BlockSpec((NC, NREL, blk, D), lambda i: (0, 0, i, 0)),
            pl.BlockSpec((NC, NREL, blk, D), lambda i: (0, 0, i, 0)),
            pl.BlockSpec((NREL, D, D), full3),
            pl.BlockSpec((NREL, D), lambda i: (0, 0)),
            pl.BlockSpec((NREL, D, D), full3),
        ],
        out_specs=[
            pl.BlockSpec((blk, D), lambda i: (i, 0)),
            pl.BlockSpec((blk, D), lambda i: (i, 0)),
            pl.BlockSpec((blk, D), lambda i: (i, 0)),
        ],
        out_shape=[
            jax.ShapeDtypeStruct((N, D), jnp.float32),
            jax.ShapeDtypeStruct((N, D), jnp.float32),
            jax.ShapeDtypeStruct((N, D), jnp.float32),
        ],
    )(xa, xe, xf, s_part, c_part, ll_w, ll_b, lr_w)


def _pad_idx(ei):
    si = jnp.concatenate([ei[0], jnp.zeros((GROWS * RL - E,), jnp.int32)])
    di = jnp.concatenate([ei[1], jnp.full((GROWS * RL - E,), N, jnp.int32)])
    return si.reshape(GROWS, RL), di.reshape(SROWS, SL)


def kernel(x_article, x_entity, x_fact, ei_article_mentions_entity,
           ei_article_supported_by_fact, ei_entity_mentioned_in_article,
           ei_entity_linked_to_fact, ei_fact_supports_entity,
           ll_w, ll_b, lr_w):
    eis = (ei_article_mentions_entity, ei_article_supported_by_fact,
           ei_entity_mentioned_in_article, ei_entity_linked_to_fact,
           ei_fact_supports_entity)
    padded = [_pad_idx(ei) for ei in eis]
    sis = [p[0] for p in padded]
    dis = [p[1] for p in padded]
    zeros_h = jnp.zeros((SL, D), jnp.float32)
    ones_h = jnp.ones((SL, D), jnp.float32)

    s_part, c_part = _make_sc_kernel()(
        x_article, x_entity, x_fact, *sis, *dis, zeros_h, ones_h)

    oa, oe, of = _tc_call(x_article, x_entity, x_fact, s_part, c_part,
                          ll_w, ll_b, lr_w)
    return (oa, oe, of)


# speedup vs baseline: 1.1119x; 1.1119x over previous
import jax
import jax.numpy as jnp
from jax import lax
from jax.experimental import pallas as pl
from jax.experimental.pallas import tpu as pltpu
import jax.experimental.pallas.tpu_sc as plsc

N = 10000
E = 320000
D = 128
NC, NS = 2, 16
NW = NC * NS
RL = 64
SL = 128
GROWS = 5120
SROWS = 2560
GCH = 16
SCH = 8
G0 = 160
G1 = (GROWS // NS) - G0
NPAD = 10240
STRIPE = NPAD // NS
NREL = 5
SRC_OF = (0, 0, 1, 1, 2)
DST_OF = (1, 2, 0, 2, 1)


def _sc_body(xa, xe, xf,
             si0, si1, si2, si3, si4,
             di0, di1, di2, di3, di4,
             zeros_h, ones_h,
             s_out, c_out,
             s_sh, siv, div, bufa, bufb, sema0, sema1, semb0, semb1):
    cid = lax.axis_index("c")
    sid = lax.axis_index("s")
    wid = cid * NS + sid
    xs = (xa, xe, xf)
    sis = (si0, si1, si2, si3, si4)
    dis = (di0, di1, di2, di3, di4)

    nfull = STRIPE // SL
    off0 = sid * STRIPE
    is0 = cid == 0
    gbase = jnp.where(is0, sid * G0, NS * G0 + sid * G1)
    sbase = jnp.where(is0, sid * (G0 // 2),
                      NS * (G0 // 2) + sid * (G1 // 2))
    nchunkw = jnp.where(is0, G0 // GCH, G1 // GCH)

    def zero_acc():
        pltpu.sync_copy(zeros_h, bufa)

        def zstep(j, carry):
            pltpu.sync_copy(bufa, s_sh.at[pl.ds(off0 + j * SL, SL)])
            return carry

        lax.fori_loop(0, nfull, zstep, 0)
        plsc.subcore_barrier()

    def flush_acc(dst):
        pltpu.sync_copy(s_sh.at[pl.ds(off0, STRIPE)],
                        dst.at[pl.ds(off0, STRIPE)])
        plsc.subcore_barrier()

    for r in range(NREL):
        src = xs[SRC_OF[r]]
        si = sis[r]
        di = dis[r]

        zero_acc()

        def gather_pair(g0, buf, s0, s1):
            da = pltpu.async_copy(src.at[siv.at[g0]], buf.at[pl.ds(0, RL)],
                                  s0)
            db = pltpu.async_copy(src.at[siv.at[g0 + 1]],
                                  buf.at[pl.ds(RL, RL)], s1)
            return da, db

        def wait_pair(buf, s0, s1):
            pltpu.make_async_copy(src.at[siv.at[0]], buf.at[pl.ds(0, RL)],
                                  s0).wait()
            pltpu.make_async_copy(src.at[siv.at[0]], buf.at[pl.ds(RL, RL)],
                                  s1).wait()

        def chunk(ch, carry):
            pltpu.sync_copy(si.at[pl.ds(gbase + ch * GCH, GCH)], siv)
            pltpu.sync_copy(di.at[pl.ds(sbase + ch * SCH, SCH)], div)
            gather_pair(0, bufa, sema0, sema1)

            def step(jj, c2):
                gather_pair(4 * jj + 2, bufb, semb0, semb1)
                wait_pair(bufa, sema0, sema1)
                pltpu.sync_copy(bufa, s_sh.at[div.at[2 * jj]], add=True)

                @pl.when(jj < SCH // 2 - 1)
                def _():
                    gather_pair(4 * jj + 4, bufa, sema0, sema1)

                wait_pair(bufb, semb0, semb1)
                pltpu.sync_copy(bufb, s_sh.at[div.at[2 * jj + 1]], add=True)
                return c2

            lax.fori_loop(0, SCH // 2, step, 0)
            return carry

        lax.fori_loop(0, nchunkw, chunk, 0)
        plsc.subcore_barrier()
        flush_acc(s_out.at[cid, r])

        zero_acc()
        pltpu.sync_copy(ones_h, bufa)

        def cchunk(ch, carry):
            pltpu.sync_copy(di.at[pl.ds(sbase + ch * SCH, SCH)], div)

            def cstep(k, c2):
                pltpu.sync_copy(bufa, s_sh.at[div.at[k]], add=True)
                return c2

            lax.fori_loop(0, SCH, cstep, 0)
            return carry

        lax.fori_loop(0, nchunkw, cchunk, 0)
        plsc.subcore_barrier()
        flush_acc(c_out.at[cid, r])


def _make_sc_kernel():
    mesh = plsc.VectorSubcoreMesh(core_axis_name="c", subcore_axis_name="s",
                                  num_cores=NC, num_subcores=NS)
    return pl.kernel(
        _sc_body,
        out_type=(
            jax.ShapeDtypeStruct((NC, NREL, NPAD, D), jnp.float32),
            jax.ShapeDtypeStruct((NC, NREL, NPAD, D), jnp.float32),
        ),
        mesh=mesh,
        scratch_types=(
            pltpu.VMEM_SHARED((NPAD, D), jnp.float32),
            pltpu.VMEM((GCH, RL), jnp.int32),
            pltpu.VMEM((SCH, SL), jnp.int32),
            pltpu.VMEM((SL, D), jnp.float32),
            pltpu.VMEM((SL, D), jnp.float32),
            pltpu.SemaphoreType.DMA,
            pltpu.SemaphoreType.DMA,
            pltpu.SemaphoreType.DMA,
            pltpu.SemaphoreType.DMA,
        ),
    )


def _tc_body(xa, xe, xf, sp, cp, llw, llb, lrw, oa, oe, of):
    x = (xa, xe, xf)
    h = []
    for r in range(NREL):
        xd = x[DST_OF[r]][...]
        s = sp[0, r] + sp[1, r]
        c = cp[0, r, :, 0] + cp[1, r, :, 0]
        c = jnp.maximum(c, 1.0)
        out = lax.dot_general(xd, llw[r], (((1,), (1,)), ((), ())),
                              precision=lax.Precision.HIGHEST,
                              preferred_element_type=jnp.float32)
        out = out + llb[r][None, :]
        agg = lax.dot_general(s, lrw[r], (((1,), (1,)), ((), ())),
                              precision=lax.Precision.HIGHEST,
                              preferred_element_type=jnp.float32)
        out = out + agg / c[:, None]
        nrm = jnp.maximum(jnp.sqrt(jnp.sum(out * out, axis=-1, keepdims=True)),
                          1e-12)
        h.append(out / nrm)
    oa[...] = h[2]
    oe[...] = h[0] + h[4]
    of[...] = h[1] + h[3]


def _tc_call(xa, xe, xf, s_part, c_part, ll_w, ll_b, lr_w):
    blk = 1280
    grid = (NPAD // blk,)
    full3 = lambda i: (0, 0, 0)
    return pl.pallas_call(
        _tc_body,
        grid=grid,
        in_specs=[
            pl.BlockSpec((blk, D), lambda i: (i, 0)),
            pl.BlockSpec((blk, D), lambda i: (i, 0)),
            pl.BlockSpec((blk, D), lambda i: (i, 0)),
            pl.BlockSpec((NC, NREL, blk, D), lambda i: (0, 0, i, 0)),
            pl.BlockSpec((NC, NREL, blk, D), lambda i: (0, 0, i, 0)),
            pl.BlockSpec((NREL, D, D), full3),
            pl.BlockSpec((NREL, D), lambda i: (0, 0)),
            pl.BlockSpec((NREL, D, D), full3),
        ],
        out_specs=[
            pl.BlockSpec((blk, D), lambda i: (i, 0)),
            pl.BlockSpec((blk, D), lambda i: (i, 0)),
            pl.BlockSpec((blk, D), lambda i: (i, 0)),
        ],
        out_shape=[
            jax.ShapeDtypeStruct((N, D), jnp.float32),
            jax.ShapeDtypeStruct((N, D), jnp.float32),
            jax.ShapeDtypeStruct((N, D), jnp.float32),
        ],
    )(xa, xe, xf, s_part, c_part, ll_w, ll_b, lr_w)


def _pad_idx(ei):
    si = jnp.concatenate([ei[0], jnp.zeros((GROWS * RL - E,), jnp.int32)])
    di = jnp.concatenate([ei[1], jnp.full((GROWS * RL - E,), N, jnp.int32)])
    return si.reshape(GROWS, RL), di.reshape(SROWS, SL)


def kernel(x_article, x_entity, x_fact, ei_article_mentions_entity,
           ei_article_supported_by_fact, ei_entity_mentioned_in_article,
           ei_entity_linked_to_fact, ei_fact_supports_entity,
           ll_w, ll_b, lr_w):
    eis = (ei_article_mentions_entity, ei_article_supported_by_fact,
           ei_entity_mentioned_in_article, ei_entity_linked_to_fact,
           ei_fact_supports_entity)
    padded = [_pad_idx(ei) for ei in eis]
    sis = [p[0] for p in padded]
    dis = [p[1] for p in padded]
    zeros_h = jnp.zeros((SL, D), jnp.float32)
    ones_h = jnp.ones((SL, D), jnp.float32)

    s_part, c_part = _make_sc_kernel()(
        x_article, x_entity, x_fact, *sis, *dis, zeros_h, ones_h)

    oa, oe, of = _tc_call(x_article, x_entity, x_fact, s_part, c_part,
                          ll_w, ll_b, lr_w)
    return (oa, oe, of)


# speedup vs baseline: 1.1643x; 1.0471x over previous
import jax
import jax.numpy as jnp
from jax import lax
from jax.experimental import pallas as pl
from jax.experimental.pallas import tpu as pltpu
import jax.experimental.pallas.tpu_sc as plsc

N = 10000
E = 320000
D = 128
NC, NS = 2, 16
NW = NC * NS
RL = 64
SL = 128
GROWS = 5120
SROWS = 2560
GCH = 16
SCH = 8
G0 = 160
G1 = (GROWS // NS) - G0
NPAD = 10240
STRIPE = NPAD // NS
NREL = 5
SRC_OF = (0, 0, 1, 1, 2)
DST_OF = (1, 2, 0, 2, 1)


def _sc_body(xa, xe, xf,
             si0, si1, si2, si3, si4,
             di0, di1, di2, di3, di4,
             zeros_h, ones_h,
             s_out, c_out,
             s_sh, siv0, div0, siv1, div1, bufa, bufb,
             sema0, sema1, semb0, semb1, semst0, semst1):
    cid = lax.axis_index("c")
    sid = lax.axis_index("s")
    wid = cid * NS + sid
    xs = (xa, xe, xf)
    sis = (si0, si1, si2, si3, si4)
    dis = (di0, di1, di2, di3, di4)

    nfull = STRIPE // SL
    off0 = sid * STRIPE
    gbase = wid * (GROWS // NW)
    sbase = wid * (SROWS // NW)
    nch2 = (GROWS // NW) // GCH // 2

    def zero_acc():
        pltpu.sync_copy(zeros_h, bufa)

        def zstep(j, carry):
            pltpu.sync_copy(bufa, s_sh.at[pl.ds(off0 + j * SL, SL)])
            return carry

        lax.fori_loop(0, nfull, zstep, 0)
        plsc.subcore_barrier()

    def flush_acc(dst):
        pltpu.sync_copy(s_sh.at[pl.ds(off0, STRIPE)],
                        dst.at[pl.ds(off0, STRIPE)])
        plsc.subcore_barrier()

    for r in range(NREL):
        src = xs[SRC_OF[r]]
        si = sis[r]
        di = dis[r]

        zero_acc()

        def issue_stage(ch, sv, dv, st, with_si):
            if with_si:
                pltpu.async_copy(si.at[pl.ds(gbase + ch * GCH, GCH)], sv, st)
            pltpu.async_copy(di.at[pl.ds(sbase + ch * SCH, SCH)], dv, st)

        def wait_stage(sv, dv, st, with_si):
            if with_si:
                pltpu.make_async_copy(si.at[pl.ds(0, GCH)], sv, st).wait()
            pltpu.make_async_copy(di.at[pl.ds(0, SCH)], dv, st).wait()

        def gather_pair(siv, g0, buf, s0, s1):
            pltpu.async_copy(src.at[siv.at[g0]], buf.at[pl.ds(0, RL)], s0)
            pltpu.async_copy(src.at[siv.at[g0 + 1]], buf.at[pl.ds(RL, RL)],
                             s1)

        def wait_pair(buf, s0, s1):
            pltpu.make_async_copy(src.at[siv0.at[0]], buf.at[pl.ds(0, RL)],
                                  s0).wait()
            pltpu.make_async_copy(src.at[siv0.at[0]], buf.at[pl.ds(RL, RL)],
                                  s1).wait()

        def run_chunk(siv, div):
            gather_pair(siv, 0, bufa, sema0, sema1)

            def step(jj, c2):
                gather_pair(siv, 4 * jj + 2, bufb, semb0, semb1)
                wait_pair(bufa, sema0, sema1)
                pltpu.sync_copy(bufa, s_sh.at[div.at[2 * jj]], add=True)

                @pl.when(jj < SCH // 2 - 1)
                def _():
                    gather_pair(siv, 4 * jj + 4, bufa, sema0, sema1)

                wait_pair(bufb, semb0, semb1)
                pltpu.sync_copy(bufb, s_sh.at[div.at[2 * jj + 1]], add=True)
                return c2

            lax.fori_loop(0, SCH // 2, step, 0)

        issue_stage(0, siv0, div0, semst0, True)

        def chunkpair(t, carry):
            wait_stage(siv0, div0, semst0, True)
            issue_stage(2 * t + 1, siv1, div1, semst1, True)
            run_chunk(siv0, div0)
            wait_stage(siv1, div1, semst1, True)

            @pl.when(t < nch2 - 1)
            def _():
                issue_stage(2 * t + 2, siv0, div0, semst0, True)

            run_chunk(siv1, div1)
            return carry

        lax.fori_loop(0, nch2, chunkpair, 0)
        plsc.subcore_barrier()
        flush_acc(s_out.at[cid, r])

        zero_acc()
        pltpu.sync_copy(ones_h, bufa)

        def crun_chunk(div):
            def cstep(k, c2):
                pltpu.sync_copy(bufa, s_sh.at[div.at[k]], add=True)
                return c2

            lax.fori_loop(0, SCH, cstep, 0)

        issue_stage(0, siv0, div0, semst0, False)

        def cchunkpair(t, carry):
            wait_stage(siv0, div0, semst0, False)
            issue_stage(2 * t + 1, siv1, div1, semst1, False)
            crun_chunk(div0)
            wait_stage(siv1, div1, semst1, False)

            @pl.when(t < nch2 - 1)
            def _():
                issue_stage(2 * t + 2, siv0, div0, semst0, False)

            crun_chunk(div1)
            return carry

        lax.fori_loop(0, nch2, cchunkpair, 0)
        plsc.subcore_barrier()
        flush_acc(c_out.at[cid, r])


def _make_sc_kernel():
    mesh = plsc.VectorSubcoreMesh(core_axis_name="c", subcore_axis_name="s",
                                  num_cores=NC, num_subcores=NS)
    return pl.kernel(
        _sc_body,
        out_type=(
            jax.ShapeDtypeStruct((NC, NREL, NPAD, D), jnp.float32),
            jax.ShapeDtypeStruct((NC, NREL, NPAD, D), jnp.float32),
        ),
        mesh=mesh,
        scratch_types=(
            pltpu.VMEM_SHARED((NPAD, D), jnp.float32),
            pltpu.VMEM((GCH, RL), jnp.int32),
            pltpu.VMEM((SCH, SL), jnp.int32),
            pltpu.VMEM((GCH, RL), jnp.int32),
            pltpu.VMEM((SCH, SL), jnp.int32),
            pltpu.VMEM((SL, D), jnp.float32),
            pltpu.VMEM((SL, D), jnp.float32),
            pltpu.SemaphoreType.DMA,
            pltpu.SemaphoreType.DMA,
            pltpu.SemaphoreType.DMA,
            pltpu.SemaphoreType.DMA,
            pltpu.SemaphoreType.DMA,
            pltpu.SemaphoreType.DMA,
        ),
    )


def _tc_body(xa, xe, xf, sp, cp, llw, llb, lrw, oa, oe, of):
    x = (xa, xe, xf)
    h = []
    for r in range(NREL):
        xd = x[DST_OF[r]][...]
        s = sp[0, r] + sp[1, r]
        c = cp[0, r, :, 0] + cp[1, r, :, 0]
        c = jnp.maximum(c, 1.0)
        out = lax.dot_general(xd, llw[r], (((1,), (1,)), ((), ())),
                              precision=lax.Precision.HIGHEST,
                              preferred_element_type=jnp.float32)
        out = out + llb[r][None, :]
        agg = lax.dot_general(s, lrw[r], (((1,), (1,)), ((), ())),
                              precision=lax.Precision.HIGHEST,
                              preferred_element_type=jnp.float32)
        out = out + agg / c[:, None]
        nrm = jnp.maximum(jnp.sqrt(jnp.sum(out * out, axis=-1, keepdims=True)),
                          1e-12)
        h.append(out / nrm)
    oa[...] = h[2]
    oe[...] = h[0] + h[4]
    of[...] = h[1] + h[3]


def _tc_call(xa, xe, xf, s_part, c_part, ll_w, ll_b, lr_w):
    blk = 1280
    grid = (NPAD // blk,)
    full3 = lambda i: (0, 0, 0)
    return pl.pallas_call(
        _tc_body,
        grid=grid,
        in_specs=[
            pl.BlockSpec((blk, D), lambda i: (i, 0)),
            pl.BlockSpec((blk, D), lambda i: (i, 0)),
            pl.BlockSpec((blk, D), lambda i: (i, 0)),
            pl.BlockSpec((NC, NREL, blk, D), lambda i: (0, 0, i, 0)),
            pl.BlockSpec((NC, NREL, blk, D), lambda i: (0, 0, i, 0)),
            pl.BlockSpec((NREL, D, D), full3),
            pl.BlockSpec((NREL, D), lambda i: (0, 0)),
            pl.BlockSpec((NREL, D, D), full3),
        ],
        out_specs=[
            pl.BlockSpec((blk, D), lambda i: (i, 0)),
            pl.BlockSpec((blk, D), lambda i: (i, 0)),
            pl.BlockSpec((blk, D), lambda i: (i, 0)),
        ],
        out_shape=[
            jax.ShapeDtypeStruct((N, D), jnp.float32),
            jax.ShapeDtypeStruct((N, D), jnp.float32),
            jax.ShapeDtypeStruct((N, D), jnp.float32),
        ],
    )(xa, xe, xf, s_part, c_part, ll_w, ll_b, lr_w)


def _pad_idx(ei):
    si = jnp.concatenate([ei[0], jnp.zeros((GROWS * RL - E,), jnp.int32)])
    di = jnp.concatenate([ei[1], jnp.full((GROWS * RL - E,), N, jnp.int32)])
    return si.reshape(GROWS, RL), di.reshape(SROWS, SL)


def kernel(x_article, x_entity, x_fact, ei_article_mentions_entity,
           ei_article_supported_by_fact, ei_entity_mentioned_in_article,
           ei_entity_linked_to_fact, ei_fact_supports_entity,
           ll_w, ll_b, lr_w):
    eis = (ei_article_mentions_entity, ei_article_supported_by_fact,
           ei_entity_mentioned_in_article, ei_entity_linked_to_fact,
           ei_fact_supports_entity)
    padded = [_pad_idx(ei) for ei in eis]
    sis = [p[0] for p in padded]
    dis = [p[1] for p in padded]
    zeros_h = jnp.zeros((SL, D), jnp.float32)
    ones_h = jnp.ones((SL, D), jnp.float32)

    s_part, c_part = _make_sc_kernel()(
        x_article, x_entity, x_fact, *sis, *dis, zeros_h, ones_h)

    oa, oe, of = _tc_call(x_article, x_entity, x_fact, s_part, c_part,
                          ll_w, ll_b, lr_w)
    return (oa, oe, of)
